# baseline (speedup 1.0000x reference)
import functools

import jax
import jax.numpy as jnp
from jax import lax
from jax.experimental import pallas as pl
from jax.experimental.pallas import tpu as pltpu
from jax.experimental.pallas import tpu_sc as plsc

_B = 4096
_M = 10
_K = 128
_VROWS = 1001
_VP = 1008
_NS = 8
_ND = 4
_F = _NS * _M

_NC = 2
_NSUB = 16
_NW = _NC * _NSUB
_BPW = _B // _NW
_L = 16
_KV = _K // _L

_mesh = plsc.VectorSubcoreMesh(core_axis_name="c", subcore_axis_name="s")


@functools.partial(
    pl.kernel,
    mesh=_mesh,
    compiler_params=pltpu.CompilerParams(needs_layout_passes=False),
    out_type=jax.ShapeDtypeStruct((_B,), jnp.float32),
    scratch_types=[
        pltpu.VMEM((_BPW * _F,), jnp.int32),
        pltpu.VMEM((_F, _K), jnp.float32),
        pltpu.VMEM((_F, _K), jnp.float32),
        pltpu.VMEM((_BPW * _ND + _L,), jnp.float32),
        pltpu.VMEM((_ND * _K,), jnp.float32),
        pltpu.VMEM((_F,), jnp.int32),
        pltpu.VMEM((_BPW * _L,), jnp.float32),
        pltpu.VMEM((_BPW,), jnp.float32),
        pltpu.SemaphoreType.DMA,
        pltpu.SemaphoreType.DMA,
    ],
)
def _fm_sc(tab_hbm, idx_hbm, den_hbm, v_hbm, off_hbm, out_hbm,
           gidx, buf0, buf1, dval, vtab, offs, totb, outv, sem0, sem1):
    wid = lax.axis_index("s") * _NC + lax.axis_index("c")

    pltpu.sync_copy(idx_hbm.at[pl.ds(wid * (_BPW * _F), _BPW * _F)], gidx)
    pltpu.sync_copy(den_hbm.at[pl.ds(wid * (_BPW * _ND), _BPW * _ND)],
                    dval.at[pl.ds(0, _BPW * _ND)])
    pltpu.sync_copy(v_hbm, vtab)
    pltpu.sync_copy(off_hbm, offs)

    offv = [offs[pl.ds(_L * j, _L)] for j in range(_F // _L)]

    def _remap(b, carry):
        base = b * _F
        for j in range(_F // _L):
            sl = pl.ds(base + _L * j, _L)
            s = gidx[sl]
            gidx[sl] = s + offv[j] + jnp.where(s == 0, _VROWS, 0)
        return carry

    lax.fori_loop(0, _BPW, _remap, 0, unroll=4)

    def _gather_start(b, buf, sem):
        pltpu.make_async_copy(
            tab_hbm.at[gidx.at[pl.ds(b * _F, _F)]], buf, sem).start()

    def _gather_wait(b, buf, sem):
        pltpu.make_async_copy(
            tab_hbm.at[gidx.at[pl.ds(b * _F, _F)]], buf, sem).wait()

    zero = jnp.zeros((_L,), jnp.float32)

    def _process(b, buf):
        def _row(r, carry):
            acc, qq = carry
            acc = list(acc)
            qq = list(qq)
            for jj in range(_KV):
                x = buf[r, pl.ds(_L * jj, _L)]
                acc[jj] = acc[jj] + x
                qq[jj] = qq[jj] + x * x
            return (tuple(acc), tuple(qq))

        init = (tuple([zero] * _KV), tuple([zero] * _KV))
        acc, qq = lax.fori_loop(0, _F, _row, init, unroll=4)
        acc = list(acc)
        qq = list(qq)
        dv = dval[pl.ds(b * _ND, _L)]
        for j in range(_ND):
            dj = jnp.full((_L,), dv[j], jnp.float32)
            for jj in range(_KV):
                t = dj * vtab[pl.ds(j * _K + _L * jj, _L)]
                acc[jj] = acc[jj] + t
                qq[jj] = qq[jj] + t * t
        tot = zero
        for jj in range(_KV):
            tot = tot + (acc[jj] * acc[jj] - qq[jj])
        totb[pl.ds(b * _L, _L)] = tot

    _gather_start(0, buf0, sem0)

    def _step(i, carry):
        b = i * 2
        _gather_start(b + 1, buf1, sem1)
        _gather_wait(b, buf0, sem0)
        _process(b, buf0)

        @pl.when(b + 2 < _BPW)
        def _():
            _gather_start(b + 2, buf0, sem0)

        _gather_wait(b + 1, buf1, sem1)
        _process(b + 1, buf1)
        return carry

    lax.fori_loop(0, _BPW // 2, _step, 0)

    lane_ids = lax.iota(jnp.int32, _L)

    def _reduce_group(g, carry):
        acc = zero
        for j in range(_L):
            col = plsc.load_gather(totb, [g * (_L * _L) + lane_ids * _L + j])
            acc = acc + col
        outv[pl.ds(g * _L, _L)] = 0.5 * acc
        return carry

    lax.fori_loop(0, _BPW // _L, _reduce_group, 0)

    pltpu.sync_copy(outv, out_hbm.at[pl.ds(wid * _BPW, _BPW)])


def kernel(s0, s1, s2, s3, s4, s5, s6, s7, d0, d1, d2, d3,
           emb0, emb1, emb2, emb3, emb4, emb5, emb6, emb7, v):
    tables = [emb0, emb1, emb2, emb3, emb4, emb5, emb6, emb7]
    tab = jnp.concatenate(
        [jnp.pad(t, ((0, _VP - _VROWS), (0, 0))) for t in tables], axis=0)
    idx = jnp.concatenate([s0, s1, s2, s3, s4, s5, s6, s7], axis=1).reshape(-1)
    dense = jnp.concatenate([d0, d1, d2, d3], axis=1).reshape(-1)
    offs = (jnp.arange(_F, dtype=jnp.int32) // _M) * _VP
    return _fm_sc(tab, idx, dense, v.reshape(-1), offs)


# speedup vs baseline: 1.2928x; 1.2928x over previous
import functools

import jax
import jax.numpy as jnp
from jax import lax
from jax.experimental import pallas as pl
from jax.experimental.pallas import tpu as pltpu
from jax.experimental.pallas import tpu_sc as plsc

_B = 4096
_M = 10
_K = 128
_VROWS = 1001
_VP = 1008
_NS = 8
_ND = 4
_F = _NS * _M

_NC = 2
_NSUB = 16
_NW = _NC * _NSUB
_BPW = _B // _NW
_L = 16
_KV = _K // _L

_mesh = plsc.VectorSubcoreMesh(core_axis_name="c", subcore_axis_name="s")


@functools.partial(
    pl.kernel,
    mesh=_mesh,
    compiler_params=pltpu.CompilerParams(needs_layout_passes=False),
    out_type=jax.ShapeDtypeStruct((_B,), jnp.float32),
    scratch_types=[
        pltpu.VMEM((_BPW * _F,), jnp.int32),
        pltpu.VMEM((_F, _K), jnp.float32),
        pltpu.VMEM((_F, _K), jnp.float32),
        pltpu.VMEM((_F, _K), jnp.float32),
        pltpu.VMEM((_F, _K), jnp.float32),
        pltpu.VMEM((_BPW * _ND + _L,), jnp.float32),
        pltpu.VMEM((_ND * _K,), jnp.float32),
        pltpu.VMEM((_F,), jnp.int32),
        pltpu.VMEM((_BPW * _L,), jnp.float32),
        pltpu.VMEM((_BPW,), jnp.float32),
        pltpu.SemaphoreType.DMA,
        pltpu.SemaphoreType.DMA,
        pltpu.SemaphoreType.DMA,
        pltpu.SemaphoreType.DMA,
    ],
)
def _fm_sc(tab_hbm, idx_hbm, den_hbm, v_hbm, off_hbm, out_hbm,
           gidx, buf0, buf1, buf2, buf3, dval, vtab, offs, totb, outv,
           sem0, sem1, sem2, sem3):
    wid = lax.axis_index("s") * _NC + lax.axis_index("c")

    pltpu.sync_copy(idx_hbm.at[pl.ds(wid * (_BPW * _F), _BPW * _F)], gidx)
    pltpu.sync_copy(den_hbm.at[pl.ds(wid * (_BPW * _ND), _BPW * _ND)],
                    dval.at[pl.ds(0, _BPW * _ND)])
    pltpu.sync_copy(v_hbm, vtab)
    pltpu.sync_copy(off_hbm, offs)

    offv = [offs[pl.ds(_L * j, _L)] for j in range(_F // _L)]

    def _remap(b, carry):
        base = b * _F
        for j in range(_F // _L):
            sl = pl.ds(base + _L * j, _L)
            s = gidx[sl]
            gidx[sl] = s + offv[j] + jnp.where(s == 0, _VROWS, 0)
        return carry

    lax.fori_loop(0, _BPW, _remap, 0, unroll=4)

    def _gather_start(b, buf, sem):
        pltpu.make_async_copy(
            tab_hbm.at[gidx.at[pl.ds(b * _F, _F)]], buf, sem).start()

    def _gather_wait(b, buf, sem):
        pltpu.make_async_copy(
            tab_hbm.at[gidx.at[pl.ds(b * _F, _F)]], buf, sem).wait()

    zero = jnp.zeros((_L,), jnp.float32)

    def _process(b, buf):
        def _row(r, carry):
            acc, qq = carry
            acc = list(acc)
            qq = list(qq)
            for jj in range(_KV):
                x = buf[r, pl.ds(_L * jj, _L)]
                acc[jj] = acc[jj] + x
                qq[jj] = qq[jj] + x * x
            return (tuple(acc), tuple(qq))

        init = (tuple([zero] * _KV), tuple([zero] * _KV))
        acc, qq = lax.fori_loop(0, _F, _row, init, unroll=4)
        acc = list(acc)
        qq = list(qq)
        dv = dval[pl.ds(b * _ND, _L)]
        for j in range(_ND):
            dj = jnp.full((_L,), dv[j], jnp.float32)
            for jj in range(_KV):
                t = dj * vtab[pl.ds(j * _K + _L * jj, _L)]
                acc[jj] = acc[jj] + t
                qq[jj] = qq[jj] + t * t
        tot = zero
        for jj in range(_KV):
            tot = tot + (acc[jj] * acc[jj] - qq[jj])
        totb[pl.ds(b * _L, _L)] = tot

    bufs = (buf0, buf1, buf2, buf3)
    sems = (sem0, sem1, sem2, sem3)
    _gather_start(0, bufs[0], sems[0])
    _gather_start(1, bufs[1], sems[1])
    _gather_start(2, bufs[2], sems[2])

    def _step(i, carry):
        for k in range(4):
            b = i * 4 + k
            nk = (k + 3) % 4

            @pl.when(b + 3 < _BPW)
            def _():
                _gather_start(b + 3, bufs[nk], sems[nk])

            _gather_wait(b, bufs[k], sems[k])
            _process(b, bufs[k])
        return carry

    lax.fori_loop(0, _BPW // 4, _step, 0)

    lane_ids = lax.iota(jnp.int32, _L)

    def _reduce_group(g, carry):
        acc = zero
        for j in range(_L):
            col = plsc.load_gather(totb, [g * (_L * _L) + lane_ids * _L + j])
            acc = acc + col
        outv[pl.ds(g * _L, _L)] = 0.5 * acc
        return carry

    lax.fori_loop(0, _BPW // _L, _reduce_group, 0)

    pltpu.sync_copy(outv, out_hbm.at[pl.ds(wid * _BPW, _BPW)])


def kernel(s0, s1, s2, s3, s4, s5, s6, s7, d0, d1, d2, d3,
           emb0, emb1, emb2, emb3, emb4, emb5, emb6, emb7, v):
    tables = [emb0, emb1, emb2, emb3, emb4, emb5, emb6, emb7]
    tab = jnp.concatenate(
        [jnp.pad(t, ((0, _VP - _VROWS), (0, 0))) for t in tables], axis=0)
    idx = jnp.concatenate([s0, s1, s2, s3, s4, s5, s6, s7], axis=1).reshape(-1)
    dense = jnp.concatenate([d0, d1, d2, d3], axis=1).reshape(-1)
    offs = (jnp.arange(_F, dtype=jnp.int32) // _M) * _VP
    return _fm_sc(tab, idx, dense, v.reshape(-1), offs)
